# baseline (speedup 1.0000x reference)
import jax
import jax.numpy as jnp
from jax import lax
from jax.experimental import pallas as pl
from jax.experimental.pallas import tpu as pltpu
from jax.experimental.pallas import tpu_sc as plsc

N = 10000
E = 320000
D = 128
H = 16
C = 3

NC = 2
NS = 16
NW = NC * NS
EPW = E // NW
CH = 1000
NCHUNK = EPW // CH
NP = 10240
ZR = NP // NS
G = 8
NG = N // G
NPG = NP // G


def _block_diag(w, rows, cols):
    tiled = jnp.tile(w, (G, G))
    r = lax.broadcasted_iota(jnp.int32, (G * rows, G * cols), 0) // rows
    c = lax.broadcasted_iota(jnp.int32, (G * rows, G * cols), 1) // cols
    return jnp.where(r == c, tiled, 0.0)


def _mm1_body(x_ref, w_ref, o_ref):
    w = _block_diag(w_ref[...], D, H)
    o_ref[...] = jnp.dot(x_ref[...], w, preferred_element_type=jnp.float32)


def _mid_body(p_ref, b_ref, o_ref):
    b = jnp.tile(b_ref[...], (1, G))
    o_ref[...] = jnp.maximum(p_ref[0] + p_ref[1] + b, 0.0)


def _fin_body(p_ref, w_ref, b_ref, o_ref):
    agg = p_ref[0, :NG] + p_ref[1, :NG]
    w = _block_diag(w_ref[...], H, C)
    b = jnp.tile(b_ref[...], (1, G))
    o_ref[...] = jnp.dot(agg, w, preferred_element_type=jnp.float32) + b


def _sc_body(y_hbm, ei_hbm, zeros_hbm, out_hbm,
             acc, src_all, dst_all, rows0, rows1, rows2,
             gsem, ssem0, ssem1, ssem2, isem):
    c = lax.axis_index("c")
    s = lax.axis_index("s")
    wid = s * NC + c
    z0 = s * ZR
    base = wid * EPW
    idx_d = []
    for k in range(NCHUNK):
        off = base + k * CH
        idx_d.append(pltpu.async_copy(ei_hbm.at[0, pl.ds(off, CH)],
                                      src_all.at[k], isem))
        idx_d.append(pltpu.async_copy(ei_hbm.at[1, pl.ds(off, CH)],
                                      dst_all.at[k], isem))
    pltpu.sync_copy(zeros_hbm, acc.at[pl.ds(z0, ZR)])
    for d in idx_d:
        d.wait()
    plsc.subcore_barrier()
    rows = (rows0, rows1, rows2)
    ssems = (ssem0, ssem1, ssem2)
    gd = [None] * NCHUNK
    sd = [None] * NCHUNK
    gd[0] = pltpu.async_copy(y_hbm.at[src_all.at[0]], rows[0], gsem)
    gd[1] = pltpu.async_copy(y_hbm.at[src_all.at[1]], rows[1], gsem)
    for k in range(NCHUNK):
        gd[k].wait()
        sd[k] = pltpu.async_copy(rows[k % 3], acc.at[dst_all.at[k]],
                                 ssems[k % 3], add=True)
        if k + 2 < NCHUNK:
            if k - 1 >= 0:
                sd[k - 1].wait()
            gd[k + 2] = pltpu.async_copy(y_hbm.at[src_all.at[k + 2]],
                                         rows[(k + 2) % 3], gsem)
    for k in range(max(0, NCHUNK - 3), NCHUNK):
        sd[k].wait()
    plsc.subcore_barrier()
    pltpu.sync_copy(acc.at[pl.ds(z0, ZR)], out_hbm.at[c, pl.ds(z0, ZR)])


_segsum = pl.kernel(
    _sc_body,
    out_type=jax.ShapeDtypeStruct((NC, NP, H), jnp.float32),
    mesh=plsc.VectorSubcoreMesh(core_axis_name="c", subcore_axis_name="s"),
    scratch_types=[
        pltpu.VMEM_SHARED((NP, H), jnp.float32),
        pltpu.VMEM((NCHUNK, CH), jnp.int32),
        pltpu.VMEM((NCHUNK, CH), jnp.int32),
        pltpu.VMEM((CH, H), jnp.float32),
        pltpu.VMEM((CH, H), jnp.float32),
        pltpu.VMEM((CH, H), jnp.float32),
        pltpu.SemaphoreType.DMA,
        pltpu.SemaphoreType.DMA,
        pltpu.SemaphoreType.DMA,
        pltpu.SemaphoreType.DMA,
        pltpu.SemaphoreType.DMA,
    ],
    compiler_params=pltpu.CompilerParams(use_tc_tiling_on_sc=False),
)


def kernel(x, edge_index, W1, b1, W2, b2):
    zeros = jnp.zeros((ZR, H), jnp.float32)

    y8 = pl.pallas_call(
        _mm1_body,
        out_shape=jax.ShapeDtypeStruct((NG, G * H), jnp.float32),
    )(x.reshape(NG, G * D), W1)
    y = y8.reshape(N, H)

    p1 = _segsum(y, edge_index, zeros)

    h8 = pl.pallas_call(
        _mid_body,
        out_shape=jax.ShapeDtypeStruct((NPG, G * H), jnp.float32),
    )(p1.reshape(NC, NPG, G * H), b1.reshape(1, H))
    h = h8.reshape(NP, H)

    p2 = _segsum(h, edge_index, zeros)

    out8 = pl.pallas_call(
        _fin_body,
        out_shape=jax.ShapeDtypeStruct((NG, G * C), jnp.float32),
    )(p2.reshape(NC, NPG, G * H), W2, b2.reshape(1, C))
    return out8.reshape(N, C)


# speedup vs baseline: 1.0041x; 1.0041x over previous
import jax
import jax.numpy as jnp
from jax import lax
from jax.experimental import pallas as pl
from jax.experimental.pallas import tpu as pltpu
from jax.experimental.pallas import tpu_sc as plsc

N = 10000
E = 320000
D = 128
H = 16
C = 3

NC = 2
NS = 16
NW = NC * NS
EPW = E // NW
CH = 2000
NCHUNK = EPW // CH
NP = 10240
ZR = NP // NS
G = 8
NG = N // G
NPG = NP // G


def _block_diag(w, rows, cols):
    tiled = jnp.tile(w, (G, G))
    r = lax.broadcasted_iota(jnp.int32, (G * rows, G * cols), 0) // rows
    c = lax.broadcasted_iota(jnp.int32, (G * rows, G * cols), 1) // cols
    return jnp.where(r == c, tiled, 0.0)


def _mm1_body(x_ref, w_ref, o_ref):
    w = _block_diag(w_ref[...], D, H)
    o_ref[...] = jnp.dot(x_ref[...], w, preferred_element_type=jnp.float32)


def _mid_body(p_ref, b_ref, o_ref):
    b = jnp.tile(b_ref[...], (1, G))
    o_ref[...] = jnp.maximum(p_ref[0] + p_ref[1] + b, 0.0)


def _fin_body(p_ref, w_ref, b_ref, o_ref):
    agg = p_ref[0, :NG] + p_ref[1, :NG]
    w = _block_diag(w_ref[...], H, C)
    b = jnp.tile(b_ref[...], (1, G))
    o_ref[...] = jnp.dot(agg, w, preferred_element_type=jnp.float32) + b


def _sc_body(y_hbm, ei_hbm, zeros_hbm, out_hbm,
             acc, src_all, dst_all, rows0, rows1, rows2,
             gsem, ssem0, ssem1, ssem2, isem):
    c = lax.axis_index("c")
    s = lax.axis_index("s")
    wid = s * NC + c
    z0 = s * ZR
    base = wid * EPW
    idx_d = []
    for k in range(NCHUNK):
        off = base + k * CH
        idx_d.append(pltpu.async_copy(ei_hbm.at[0, pl.ds(off, CH)],
                                      src_all.at[k], isem))
        idx_d.append(pltpu.async_copy(ei_hbm.at[1, pl.ds(off, CH)],
                                      dst_all.at[k], isem))
    pltpu.sync_copy(zeros_hbm, acc.at[pl.ds(z0, ZR)])
    for d in idx_d:
        d.wait()
    plsc.subcore_barrier()
    rows = (rows0, rows1, rows2)
    ssems = (ssem0, ssem1, ssem2)
    gd = [None] * NCHUNK
    sd = [None] * NCHUNK
    gd[0] = pltpu.async_copy(y_hbm.at[src_all.at[0]], rows[0], gsem)
    gd[1] = pltpu.async_copy(y_hbm.at[src_all.at[1]], rows[1], gsem)
    for k in range(NCHUNK):
        gd[k].wait()
        sd[k] = pltpu.async_copy(rows[k % 3], acc.at[dst_all.at[k]],
                                 ssems[k % 3], add=True)
        if k + 2 < NCHUNK:
            if k - 1 >= 0:
                sd[k - 1].wait()
            gd[k + 2] = pltpu.async_copy(y_hbm.at[src_all.at[k + 2]],
                                         rows[(k + 2) % 3], gsem)
    for k in range(max(0, NCHUNK - 3), NCHUNK):
        sd[k].wait()
    plsc.subcore_barrier()
    pltpu.sync_copy(acc.at[pl.ds(z0, ZR)], out_hbm.at[c, pl.ds(z0, ZR)])


_segsum = pl.kernel(
    _sc_body,
    out_type=jax.ShapeDtypeStruct((NC, NP, H), jnp.float32),
    mesh=plsc.VectorSubcoreMesh(core_axis_name="c", subcore_axis_name="s"),
    scratch_types=[
        pltpu.VMEM_SHARED((NP, H), jnp.float32),
        pltpu.VMEM((NCHUNK, CH), jnp.int32),
        pltpu.VMEM((NCHUNK, CH), jnp.int32),
        pltpu.VMEM((CH, H), jnp.float32),
        pltpu.VMEM((CH, H), jnp.float32),
        pltpu.VMEM((CH, H), jnp.float32),
        pltpu.SemaphoreType.DMA,
        pltpu.SemaphoreType.DMA,
        pltpu.SemaphoreType.DMA,
        pltpu.SemaphoreType.DMA,
        pltpu.SemaphoreType.DMA,
    ],
    compiler_params=pltpu.CompilerParams(use_tc_tiling_on_sc=False),
)


def kernel(x, edge_index, W1, b1, W2, b2):
    zeros = jnp.zeros((ZR, H), jnp.float32)

    y8 = pl.pallas_call(
        _mm1_body,
        out_shape=jax.ShapeDtypeStruct((NG, G * H), jnp.float32),
    )(x.reshape(NG, G * D), W1)
    y = y8.reshape(N, H)

    p1 = _segsum(y, edge_index, zeros)

    h8 = pl.pallas_call(
        _mid_body,
        out_shape=jax.ShapeDtypeStruct((NPG, G * H), jnp.float32),
    )(p1.reshape(NC, NPG, G * H), b1.reshape(1, H))
    h = h8.reshape(NP, H)

    p2 = _segsum(h, edge_index, zeros)

    out8 = pl.pallas_call(
        _fin_body,
        out_shape=jax.ShapeDtypeStruct((NG, G * C), jnp.float32),
    )(p2.reshape(NC, NPG, G * H), W2, b2.reshape(1, C))
    return out8.reshape(N, C)


# speedup vs baseline: 1.0551x; 1.0508x over previous
import jax
import jax.numpy as jnp
from jax import lax
from jax.experimental import pallas as pl
from jax.experimental.pallas import tpu as pltpu
from jax.experimental.pallas import tpu_sc as plsc

N = 10000
E = 320000
D = 128
H = 16
C = 3

NC = 2
NS = 16
NW = NC * NS
EPW = E // NW
CH = 2000
NCHUNK = EPW // CH
NP = 10240
ZR = NP // NS
G = 8
NG = N // G
NPG = NP // G


def _block_diag(w, rows, cols):
    tiled = jnp.tile(w, (G, G))
    r = lax.broadcasted_iota(jnp.int32, (G * rows, G * cols), 0) // rows
    c = lax.broadcasted_iota(jnp.int32, (G * rows, G * cols), 1) // cols
    return jnp.where(r == c, tiled, 0.0)


def _mm1_body(x_ref, w_ref, o_ref):
    w = _block_diag(w_ref[...], D, H)
    o_ref[...] = jnp.dot(x_ref[...], w, preferred_element_type=jnp.float32)


def _mid_body(p_ref, b_ref, o_ref):
    b = jnp.tile(b_ref[...], (1, G))
    o_ref[...] = jnp.maximum(p_ref[0] + p_ref[1] + b, 0.0)


def _fin_body(p_ref, w_ref, b_ref, o_ref):
    agg = p_ref[0, :NG] + p_ref[1, :NG]
    w = _block_diag(w_ref[...], H, C)
    b = jnp.tile(b_ref[...], (1, G))
    o_ref[...] = jnp.dot(agg, w, preferred_element_type=jnp.float32) + b


def _sc_body(y_hbm, ei_hbm, zeros_hbm, out_hbm,
             acc, src_all, dst_all, rows0, rows1, rows2,
             gsem, ssem0, ssem1, ssem2, isem):
    c = lax.axis_index("c")
    s = lax.axis_index("s")
    wid = s * NC + c
    z0 = s * ZR
    base = wid * EPW
    idx_d = []
    for k in range(NCHUNK):
        off = base + k * CH
        idx_d.append(pltpu.async_copy(ei_hbm.at[0, pl.ds(off, CH)],
                                      src_all.at[k], isem))
        idx_d.append(pltpu.async_copy(ei_hbm.at[1, pl.ds(off, CH)],
                                      dst_all.at[k], isem))
    pltpu.sync_copy(zeros_hbm.at[pl.ds(z0, ZR)], acc.at[pl.ds(z0, ZR)])
    for d in idx_d:
        d.wait()
    plsc.subcore_barrier()
    rows = (rows0, rows1, rows2)
    ssems = (ssem0, ssem1, ssem2)
    gd = [None] * NCHUNK
    sd = [None] * NCHUNK
    gd[0] = pltpu.async_copy(y_hbm.at[src_all.at[0]], rows[0], gsem)
    gd[1] = pltpu.async_copy(y_hbm.at[src_all.at[1]], rows[1], gsem)
    for k in range(NCHUNK):
        gd[k].wait()
        sd[k] = pltpu.async_copy(rows[k % 3], acc.at[dst_all.at[k]],
                                 ssems[k % 3], add=True)
        if k + 2 < NCHUNK:
            if k - 1 >= 0:
                sd[k - 1].wait()
            gd[k + 2] = pltpu.async_copy(y_hbm.at[src_all.at[k + 2]],
                                         rows[(k + 2) % 3], gsem)
    for k in range(max(0, NCHUNK - 3), NCHUNK):
        sd[k].wait()
    plsc.subcore_barrier()
    pltpu.sync_copy(acc.at[pl.ds(z0, ZR)], out_hbm.at[c, pl.ds(z0, ZR)])


_segsum = pl.kernel(
    _sc_body,
    out_type=jax.ShapeDtypeStruct((NC, NP, H), jnp.float32),
    mesh=plsc.VectorSubcoreMesh(core_axis_name="c", subcore_axis_name="s"),
    scratch_types=[
        pltpu.VMEM_SHARED((NP, H), jnp.float32),
        pltpu.VMEM((NCHUNK, CH), jnp.int32),
        pltpu.VMEM((NCHUNK, CH), jnp.int32),
        pltpu.VMEM((CH, H), jnp.float32),
        pltpu.VMEM((CH, H), jnp.float32),
        pltpu.VMEM((CH, H), jnp.float32),
        pltpu.SemaphoreType.DMA,
        pltpu.SemaphoreType.DMA,
        pltpu.SemaphoreType.DMA,
        pltpu.SemaphoreType.DMA,
        pltpu.SemaphoreType.DMA,
    ],
    compiler_params=pltpu.CompilerParams(use_tc_tiling_on_sc=False),
)


def kernel(x, edge_index, W1, b1, W2, b2):
    zeros = jnp.zeros((NP, H), jnp.float32)

    y8 = pl.pallas_call(
        _mm1_body,
        out_shape=jax.ShapeDtypeStruct((NG, G * H), jnp.float32),
    )(x.reshape(NG, G * D), W1)
    y = y8.reshape(N, H)

    p1 = _segsum(y, edge_index, zeros)

    h8 = pl.pallas_call(
        _mid_body,
        out_shape=jax.ShapeDtypeStruct((NPG, G * H), jnp.float32),
    )(p1.reshape(NC, NPG, G * H), b1.reshape(1, H))
    h = h8.reshape(NP, H)

    p2 = _segsum(h, edge_index, zeros)

    out8 = pl.pallas_call(
        _fin_body,
        out_shape=jax.ShapeDtypeStruct((NG, G * C), jnp.float32),
    )(p2.reshape(NC, NPG, G * H), W2, b2.reshape(1, C))
    return out8.reshape(N, C)


# speedup vs baseline: 1.0635x; 1.0080x over previous
import jax
import jax.numpy as jnp
from jax import lax
from jax.experimental import pallas as pl
from jax.experimental.pallas import tpu as pltpu
from jax.experimental.pallas import tpu_sc as plsc

N = 10000
E = 320000
D = 128
H = 16
C = 3

NC = 2
NS = 16
NW = NC * NS
EPW = E // NW
CH = 1000
NCHUNK = EPW // CH
NBUF = 4
NP = 10240
ZR = NP // NS
G = 8
NG = N // G
NPG = NP // G


def _block_diag(w, rows, cols):
    tiled = jnp.tile(w, (G, G))
    r = lax.broadcasted_iota(jnp.int32, (G * rows, G * cols), 0) // rows
    c = lax.broadcasted_iota(jnp.int32, (G * rows, G * cols), 1) // cols
    return jnp.where(r == c, tiled, 0.0)


def _mm1_body(x_ref, w_ref, o_ref):
    w = _block_diag(w_ref[...], D, H)
    o_ref[...] = jnp.dot(x_ref[...], w, preferred_element_type=jnp.float32)


def _mid_body(p_ref, b_ref, o_ref):
    b = jnp.tile(b_ref[...], (1, G))
    o_ref[...] = jnp.maximum(p_ref[0] + p_ref[1] + b, 0.0)


def _fin_body(p_ref, w_ref, b_ref, o_ref):
    agg = p_ref[0, :NG] + p_ref[1, :NG]
    w = _block_diag(w_ref[...], H, C)
    b = jnp.tile(b_ref[...], (1, G))
    o_ref[...] = jnp.dot(agg, w, preferred_element_type=jnp.float32) + b


def _sc_body(y_hbm, ei_hbm, zeros_hbm, out_hbm,
             acc, src_all, dst_all, rows0, rows1, rows2, rows3,
             gsem, ssem0, ssem1, ssem2, ssem3, isem):
    c = lax.axis_index("c")
    s = lax.axis_index("s")
    wid = s * NC + c
    z0 = s * ZR
    base = wid * EPW
    idx_d = []
    for k in range(NCHUNK):
        off = base + k * CH
        idx_d.append(pltpu.async_copy(ei_hbm.at[0, pl.ds(off, CH)],
                                      src_all.at[k], isem))
        idx_d.append(pltpu.async_copy(ei_hbm.at[1, pl.ds(off, CH)],
                                      dst_all.at[k], isem))
    pltpu.sync_copy(zeros_hbm.at[pl.ds(z0, ZR)], acc.at[pl.ds(z0, ZR)])
    for d in idx_d:
        d.wait()
    plsc.subcore_barrier()
    rows = (rows0, rows1, rows2, rows3)[:NBUF]
    ssems = (ssem0, ssem1, ssem2, ssem3)[:NBUF]
    gd = [None] * NCHUNK
    sd = [None] * NCHUNK
    for k in range(min(NBUF - 1, NCHUNK)):
        gd[k] = pltpu.async_copy(y_hbm.at[src_all.at[k]], rows[k], gsem)
    for k in range(NCHUNK):
        gd[k].wait()
        sd[k] = pltpu.async_copy(rows[k % NBUF], acc.at[dst_all.at[k]],
                                 ssems[k % NBUF], add=True)
        nxt = k + NBUF - 1
        if nxt < NCHUNK:
            if k - 1 >= 0:
                sd[k - 1].wait()
            gd[nxt] = pltpu.async_copy(y_hbm.at[src_all.at[nxt]],
                                       rows[nxt % NBUF], gsem)
    for k in range(max(0, NCHUNK - NBUF), NCHUNK):
        sd[k].wait()
    plsc.subcore_barrier()
    pltpu.sync_copy(acc.at[pl.ds(z0, ZR)], out_hbm.at[c, pl.ds(z0, ZR)])


_segsum = pl.kernel(
    _sc_body,
    out_type=jax.ShapeDtypeStruct((NC, NP, H), jnp.float32),
    mesh=plsc.VectorSubcoreMesh(core_axis_name="c", subcore_axis_name="s"),
    scratch_types=[
        pltpu.VMEM_SHARED((NP, H), jnp.float32),
        pltpu.VMEM((NCHUNK, CH), jnp.int32),
        pltpu.VMEM((NCHUNK, CH), jnp.int32),
        pltpu.VMEM((CH, H), jnp.float32),
        pltpu.VMEM((CH, H), jnp.float32),
        pltpu.VMEM((CH, H), jnp.float32),
        pltpu.VMEM((CH, H), jnp.float32),
        pltpu.SemaphoreType.DMA,
        pltpu.SemaphoreType.DMA,
        pltpu.SemaphoreType.DMA,
        pltpu.SemaphoreType.DMA,
        pltpu.SemaphoreType.DMA,
        pltpu.SemaphoreType.DMA,
    ],
    compiler_params=pltpu.CompilerParams(use_tc_tiling_on_sc=False),
)


def kernel(x, edge_index, W1, b1, W2, b2):
    zeros = jnp.zeros((NP, H), jnp.float32)

    y8 = pl.pallas_call(
        _mm1_body,
        out_shape=jax.ShapeDtypeStruct((NG, G * H), jnp.float32),
    )(x.reshape(NG, G * D), W1)
    y = y8.reshape(N, H)

    p1 = _segsum(y, edge_index, zeros)

    h8 = pl.pallas_call(
        _mid_body,
        out_shape=jax.ShapeDtypeStruct((NPG, G * H), jnp.float32),
    )(p1.reshape(NC, NPG, G * H), b1.reshape(1, H))
    h = h8.reshape(NP, H)

    p2 = _segsum(h, edge_index, zeros)

    out8 = pl.pallas_call(
        _fin_body,
        out_shape=jax.ShapeDtypeStruct((NG, G * C), jnp.float32),
    )(p2.reshape(NC, NPG, G * H), W2, b2.reshape(1, C))
    return out8.reshape(N, C)
